# baseline (speedup 1.0000x reference)
import functools

import jax
import jax.numpy as jnp
from jax import lax
from jax.experimental import pallas as pl
from jax.experimental.pallas import tpu as pltpu
from jax.experimental.pallas import tpu_sc as plsc

BATCH = 1024
SEQ = 50
NROWS = BATCH * SEQ
HIDDEN = 64
NW = 32
ROWS_PER_W = NROWS // NW
CHUNK = 800
RB = 3200


def _gather_embeddings(len2, ipd2, pidx_len, pidx_ipd, nrows):
    mesh = plsc.VectorSubcoreMesh(core_axis_name="c", subcore_axis_name="s")
    rows_per_w = nrows // NW

    @functools.partial(
        pl.kernel,
        mesh=mesh,
        out_type=(
            jax.ShapeDtypeStruct((nrows, 2 * HIDDEN), jnp.float32),
            jax.ShapeDtypeStruct((nrows, 2 * HIDDEN), jnp.float32),
        ),
        scratch_types=[
            pltpu.VMEM((CHUNK,), jnp.int32),
            pltpu.VMEM((CHUNK, 2 * HIDDEN), jnp.float32),
            pltpu.SemaphoreType.DMA,
        ],
    )
    def k(len_hbm, ipd_hbm, il_hbm, ii_hbm, out_l, out_i, idx_v, rows_v, sem):
        wid = lax.axis_index("s") * 2 + lax.axis_index("c")
        base0 = wid * rows_per_w
        for t_hbm, i_hbm, o_hbm in ((len_hbm, il_hbm, out_l),
                                    (ipd_hbm, ii_hbm, out_i)):
            for ch in range(rows_per_w // CHUNK):
                base = base0 + ch * CHUNK
                pltpu.sync_copy(i_hbm.at[pl.ds(base, CHUNK)], idx_v)
                pltpu.async_copy(t_hbm.at[idx_v], rows_v, sem).wait()
                pltpu.sync_copy(rows_v, o_hbm.at[pl.ds(base, CHUNK)])

    return k(len2, ipd2, pidx_len, pidx_ipd)


def _build_mats(S, H, T, LUT):
    eye = jnp.eye(8, dtype=jnp.float32)
    St = jnp.pad(jnp.transpose(S, (0, 2, 1)), ((0, 0), (0, 1), (0, 0)))
    M1 = (eye[:, None, :, None] * St[:, :, None, :]).reshape(128, 64)
    Tc = jnp.pad(T, ((0, 0), (0, 1))).reshape(128, 1)
    Hp = jnp.pad(H.T, ((0, 0), (0, 1)))
    M2 = (eye[:, None, :, None] * Hp[None, :, None, :]).reshape(128, 128)
    M3 = jnp.transpose(LUT, (2, 0, 1)).reshape(LUT.shape[2], 128)
    return M1, Tc, M2, M3


def _pq_stage(vT_or_v, M1, Tc, M2, n, transposed_in=True):
    if transposed_in:
        pT = jnp.dot(M1, vT_or_v)
    else:
        pT = lax.dot_general(M1, vT_or_v, (((1,), (1,)), ((), ())))
    sT = jnp.where(pT - Tc > 0, 1.0, -1.0)
    lT = jnp.dot(M2, sT)
    v = lT.reshape(8, 16, n)
    m = v
    for hw in (8, 4, 2, 1):
        m = jnp.maximum(m[:, :hw, :], m[:, hw:2 * hw, :])
    iot = lax.broadcasted_iota(jnp.int32, (8, 16, n), 1)
    cand = jnp.where(v == m, iot, 16)
    mi = cand
    for hw in (8, 4, 2, 1):
        mi = jnp.minimum(mi[:, :hw, :], mi[:, hw:2 * hw, :])
    return (iot == mi).astype(jnp.float32).reshape(128, n)


RBF = 5 * BATCH
HGRID = NROWS // RBF // 2


def _make_half_body(final_half):
    def body(lg_ref, ig_ref, parl_ref, pari_ref,
             Mx1_ref, Txc_ref, Mx2_ref, Mx3_ref,
             Mh1_ref, Thc_ref, Mh2_ref, Mh3_ref,
             Mf1_ref, Tfc_ref, Mf2_ref, Mf3_ref, h0_ref,
             out_ref, hT_ref, AT_ref):
        i = pl.program_id(0)

        @pl.when(i == 0)
        def _():
            if final_half:
                hT_ref[...] = h0_ref[...]
            else:
                hT_ref[...] = jnp.zeros((HIDDEN, BATCH), jnp.float32)

        lgv = lg_ref[...]
        igv = ig_ref[...]
        len_sel = jnp.where(parl_ref[...] == 1, lgv[:, HIDDEN:],
                            lgv[:, :HIDDEN])
        ipd_sel = jnp.where(pari_ref[...] == 1, igv[:, HIDDEN:],
                            igv[:, :HIDDEN])
        xe = ipd_sel + len_sel
        oh = _pq_stage(xe, Mx1_ref[...], Txc_ref[...], Mx2_ref[...], RBF,
                       transposed_in=False)
        AT_ref[...] = jnp.dot(Mx3_ref[...], oh)

        Mh1 = Mh1_ref[...]
        Thc = Thc_ref[...]
        Mh2 = Mh2_ref[...]
        Mh3 = Mh3_ref[...]
        for k in range(RBF // BATCH):
            at = AT_ref[:, k * BATCH:(k + 1) * BATCH]
            ohk = _pq_stage(hT_ref[...], Mh1, Thc, Mh2, BATCH)
            hT_ref[...] = at + jnp.dot(Mh3, ohk)

        @pl.when(i == HGRID - 1)
        def _():
            if final_half:
                ohf = _pq_stage(hT_ref[...], Mf1_ref[...], Tfc_ref[...],
                                Mf2_ref[...], BATCH)
                o = lax.dot_general(ohf, Mf3_ref[...],
                                    (((0,), (1,)), ((), ())))
                m = jnp.max(o, axis=-1, keepdims=True)
                sh = o - m
                out_ref[...] = sh - jnp.log(jnp.sum(jnp.exp(sh), axis=-1,
                                                    keepdims=True))
            else:
                out_ref[...] = hT_ref[...]

    return body


def _fused_half(lg, ig, parl, pari, mats_x, mats_h, mats_f, h0, final_half):
    wspec = [
        pl.BlockSpec((128, 64), lambda i: (0, 0)),
        pl.BlockSpec((128, 1), lambda i: (0, 0)),
        pl.BlockSpec((128, 128), lambda i: (0, 0)),
    ]
    if final_half:
        out_spec = pl.BlockSpec((BATCH, 100), lambda i: (0, 0))
        out_shape = jax.ShapeDtypeStruct((BATCH, 100), jnp.float32)
    else:
        out_spec = pl.BlockSpec((HIDDEN, BATCH), lambda i: (0, 0))
        out_shape = jax.ShapeDtypeStruct((HIDDEN, BATCH), jnp.float32)
    return pl.pallas_call(
        _make_half_body(final_half),
        grid=(HGRID,),
        in_specs=[
            pl.BlockSpec((RBF, 2 * HIDDEN), lambda i: (i, 0)),
            pl.BlockSpec((RBF, 2 * HIDDEN), lambda i: (i, 0)),
            pl.BlockSpec((RBF, 1), lambda i: (i, 0)),
            pl.BlockSpec((RBF, 1), lambda i: (i, 0)),
        ] + wspec + [pl.BlockSpec((64, 128), lambda i: (0, 0))]
          + wspec + [pl.BlockSpec((64, 128), lambda i: (0, 0))]
          + wspec + [pl.BlockSpec((100, 128), lambda i: (0, 0))]
          + [pl.BlockSpec((HIDDEN, BATCH), lambda i: (0, 0))],
        out_specs=out_spec,
        out_shape=out_shape,
        scratch_shapes=[pltpu.VMEM((HIDDEN, BATCH), jnp.float32),
                        pltpu.VMEM((HIDDEN, RBF), jnp.float32)],
    )(lg, ig, parl, pari, *mats_x, *mats_h, *mats_f, h0)


def kernel(x, S1, H1, T1, LUT1, S2, H2, T2, LUT2, lenLUT, ipdLUT):
    xt = x.astype(jnp.int32).transpose(1, 0, 2)
    idx_len = xt[:, :, 0].reshape(NROWS)
    idx_ipd = xt[:, :, 1].reshape(NROWS)

    len2 = lenLUT.reshape(-1, 2 * HIDDEN)
    ipd2 = ipdLUT.reshape(-1, 2 * HIDDEN)
    half = NROWS // 2
    lg1, ig1 = _gather_embeddings(len2, ipd2, (idx_len >> 1)[:half],
                                  (idx_ipd >> 1)[:half], half)
    lg2, ig2 = _gather_embeddings(len2, ipd2, (idx_len >> 1)[half:],
                                  (idx_ipd >> 1)[half:], half)
    parl = (idx_len & 1).reshape(NROWS, 1)
    pari = (idx_ipd & 1).reshape(NROWS, 1)

    mats_x = _build_mats(S1[:8], H1, T1[:8], LUT1[:8])
    mats_h = _build_mats(S1[8:], H1, T1[8:], LUT1[8:])
    mats_f = _build_mats(S2, H2, T2, LUT2)

    h0 = jnp.zeros((HIDDEN, BATCH), jnp.float32)
    hmid = _fused_half(lg1, ig1, parl[:half], pari[:half],
                       mats_x, mats_h, mats_f, h0, final_half=False)
    return _fused_half(lg2, ig2, parl[half:], pari[half:],
                       mats_x, mats_h, mats_f, hmid, final_half=True)
